# baseline (speedup 1.0000x reference)
import jax
import jax.numpy as jnp
from jax import lax
from jax.experimental import pallas as pl
from jax.experimental.pallas import tpu as pltpu
from jax.experimental.pallas import tpu_sc as plsc

N = 10000
E = 320000
D = 128

CHUNK = 128
N_CHUNKS = E // CHUNK
NW = 32
CH_PER_W = N_CHUNKS // NW
CH_REM = N_CHUNKS % NW
PIPE = 2
FULL_ITERS = CH_PER_W // PIPE
N_PAD = 10240
ROWS_PER_TILE = N_PAD // 16
DEG_PAD = 10240
DEG_PER_TILE = DEG_PAD // 16


def _make_sc_agg(with_deg):
    mesh = plsc.VectorSubcoreMesh(core_axis_name="c", subcore_axis_name="s")
    out_type = [jax.ShapeDtypeStruct((2, N_PAD, D), jnp.float32)]
    scratch = (
        [pltpu.VMEM((CHUNK,), jnp.int32) for _ in range(PIPE)]
        + [pltpu.VMEM((CHUNK,), jnp.int32) for _ in range(PIPE)]
        + [
            pltpu.VMEM((PIPE, CHUNK, D), jnp.float32),
            pltpu.VMEM_SHARED((N_PAD, D), jnp.float32),
        ]
        + [pltpu.SemaphoreType.DMA for _ in range(5 * PIPE)]
    )
    if with_deg:
        out_type.append(jax.ShapeDtypeStruct((2, DEG_PAD), jnp.float32))
        scratch += [
            pltpu.VMEM((CHUNK,), jnp.float32),
            pltpu.VMEM((DEG_PER_TILE,), jnp.float32),
            pltpu.VMEM_SHARED((DEG_PAD,), jnp.float32),
        ]

    def body(src_hbm, dst_hbm, x_hbm, *refs):
        if with_deg:
            agg_out, deg_out = refs[:2]
            refs = refs[2:]
        else:
            agg_out = refs[0]
            refs = refs[1:]
        srcs = refs[:PIPE]
        dsts = refs[PIPE:2 * PIPE]
        rows_v = refs[2 * PIPE]
        agg_sh = refs[2 * PIPE + 1]
        sems = refs[2 * PIPE + 2:2 * PIPE + 2 + 5 * PIPE]
        isem_s = sems[:PIPE]
        isem_d = sems[PIPE:2 * PIPE]
        gsem = sems[2 * PIPE:3 * PIPE]
        ssem = sems[3 * PIPE:4 * PIPE]
        dsem = sems[4 * PIPE:]
        if with_deg:
            ones_v, zdeg_v, deg_sh = refs[2 * PIPE + 2 + 5 * PIPE:]
        cid = lax.axis_index("c")
        sid = lax.axis_index("s")
        wid = sid * 2 + cid
        start_chunk = wid * CH_PER_W + jnp.minimum(wid, CH_REM)
        n_my = CH_PER_W + jnp.where(wid < CH_REM, 1, 0)

        def zb(i, _):
            rows_v[0, i // 8, pl.ds((i % 8) * 16, 16)] = jnp.zeros((16,), jnp.float32)
            return 0
        lax.fori_loop(0, CHUNK * 8, zb, 0)
        for z in range(ROWS_PER_TILE // CHUNK):
            pltpu.sync_copy(
                rows_v.at[0], agg_sh.at[pl.ds(sid * ROWS_PER_TILE + z * CHUNK, CHUNK)])
        if with_deg:
            def zd(i, _):
                zdeg_v[pl.ds(i * 16, 16)] = jnp.zeros((16,), jnp.float32)
                ones_v[pl.ds((i % 4) * 16, 16)] = jnp.ones((16,), jnp.float32)
                return 0
            lax.fori_loop(0, DEG_PER_TILE // 16, zd, 0)
            pltpu.sync_copy(zdeg_v, deg_sh.at[pl.ds(sid * DEG_PER_TILE, DEG_PER_TILE)])
        plsc.subcore_barrier()

        def drain_scatter(k):
            pltpu.make_async_copy(rows_v.at[k], agg_sh.at[dsts[k]], ssem[k]).wait()
            if with_deg:
                pltpu.make_async_copy(ones_v, deg_sh.at[dsts[k]], dsem[k]).wait()

        def pipe_body(t, _):
            iids = []
            for k in range(PIPE):
                @pl.when(t > 0)
                def _(k=k):
                    drain_scatter(k)
                base = (start_chunk + t * PIPE + k) * CHUNK
                iids.append((
                    pltpu.async_copy(src_hbm.at[pl.ds(base, CHUNK)], srcs[k], isem_s[k]),
                    pltpu.async_copy(dst_hbm.at[pl.ds(base, CHUNK)], dsts[k], isem_d[k]),
                ))
            gds = []
            for k in range(PIPE):
                iids[k][0].wait()
                gds.append(pltpu.async_copy(x_hbm.at[srcs[k]], rows_v.at[k], gsem[k]))
            for k in range(PIPE):
                gds[k].wait()
                iids[k][1].wait()
                pltpu.async_copy(rows_v.at[k], agg_sh.at[dsts[k]], ssem[k], add=True)
                if with_deg:
                    pltpu.async_copy(ones_v, deg_sh.at[dsts[k]], dsem[k], add=True)
            return 0
        lax.fori_loop(0, FULL_ITERS, pipe_body, 0)
        for k in range(PIPE):
            drain_scatter(k)

        def tail_body(j, _):
            base = (start_chunk + j) * CHUNK
            pltpu.async_copy(src_hbm.at[pl.ds(base, CHUNK)], srcs[0], isem_s[0]).wait()
            pltpu.async_copy(dst_hbm.at[pl.ds(base, CHUNK)], dsts[0], isem_d[0]).wait()
            pltpu.async_copy(x_hbm.at[srcs[0]], rows_v.at[0], gsem[0]).wait()
            pltpu.sync_copy(rows_v.at[0], agg_sh.at[dsts[0]], add=True)
            if with_deg:
                pltpu.sync_copy(ones_v, deg_sh.at[dsts[0]], add=True)
            return 0
        lax.fori_loop(FULL_ITERS * PIPE, n_my, tail_body, 0)
        plsc.subcore_barrier()

        rbase = sid * ROWS_PER_TILE
        pltpu.sync_copy(agg_sh.at[pl.ds(rbase, ROWS_PER_TILE)],
                        agg_out.at[cid, pl.ds(rbase, ROWS_PER_TILE)])
        if with_deg:
            dbase = sid * DEG_PER_TILE
            pltpu.sync_copy(deg_sh.at[pl.ds(dbase, DEG_PER_TILE)],
                            deg_out.at[cid, pl.ds(dbase, DEG_PER_TILE)])

    return pl.kernel(body, out_type=out_type, mesh=mesh, scratch_types=scratch)


_sc_agg_deg = _make_sc_agg(True)
_sc_agg = _make_sc_agg(False)

BR = 1024

_DN = (((1,), (1,)), ((), ()))


def _make_dense_r():
    def body(x_ref, wr_ref, o_ref):
        o_ref[...] = lax.dot_general(x_ref[...], wr_ref[...], _DN,
                                     preferred_element_type=jnp.float32)

    return pl.pallas_call(
        body,
        grid=(N_PAD // BR,),
        in_specs=[
            pl.BlockSpec((BR, D), lambda i: (i, 0)),
            pl.BlockSpec((D, D), lambda i: (0, 0)),
        ],
        out_specs=pl.BlockSpec((BR, D), lambda i: (i, 0)),
        out_shape=jax.ShapeDtypeStruct((N_PAD, D), jnp.float32),
    )


def _make_combine(final_layer):
    def body(p0_ref, p1_ref, d0_ref, d1_ref, xr_ref, wl_ref, b_ref, o_ref):
        deg = jnp.maximum(d0_ref[0, 0] + d1_ref[0, 0], 1.0)
        agg = (p0_ref[0] + p1_ref[0]) / deg
        hl = lax.dot_general(agg, wl_ref[...], _DN, preferred_element_type=jnp.float32)
        h = hl + xr_ref[...] + b_ref[...]
        if final_layer:
            nrm = jnp.sqrt(jnp.sum(h * h, axis=1, keepdims=True))
            o_ref[...] = h / jnp.maximum(nrm, 1e-12)
        else:
            o_ref[...] = jnp.maximum(h, 0.0)

    return pl.pallas_call(
        body,
        grid=(N_PAD // BR,),
        in_specs=[
            pl.BlockSpec((1, BR, D), lambda i: (0, i, 0)),
            pl.BlockSpec((1, BR, D), lambda i: (1, i, 0)),
            pl.BlockSpec((1, 1, BR, 1), lambda i: (0, i, 0, 0)),
            pl.BlockSpec((1, 1, BR, 1), lambda i: (1, i, 0, 0)),
            pl.BlockSpec((BR, D), lambda i: (i, 0)),
            pl.BlockSpec((D, D), lambda i: (0, 0)),
            pl.BlockSpec((1, D), lambda i: (0, 0)),
        ],
        out_specs=pl.BlockSpec((BR, D), lambda i: (i, 0)),
        out_shape=jax.ShapeDtypeStruct((N_PAD, D), jnp.float32),
    )


_dense_r = _make_dense_r()
_combine1 = _make_combine(False)
_combine2 = _make_combine(True)


def kernel(x, edge_index, W1l, b1l, W1r, W2l, b2l, W2r):
    src = edge_index[0]
    dst = edge_index[1]
    xp = jnp.pad(x, ((0, N_PAD - N), (0, 0)))
    part1, degp = _sc_agg_deg(src, dst, xp)
    xr1 = _dense_r(xp, W1r)
    dd = degp.reshape(2, N_PAD // BR, BR, 1)
    h = _combine1(part1, part1, dd, dd, xr1, W1l, b1l.reshape(1, D))
    (part2,) = _sc_agg(src, dst, h)
    xr2 = _dense_r(h, W2r)
    out = _combine2(part2, part2, dd, dd, xr2, W2l, b2l.reshape(1, D))
    return out[:N]


# speedup vs baseline: 1.0515x; 1.0515x over previous
import jax
import jax.numpy as jnp
from jax import lax
from jax.experimental import pallas as pl
from jax.experimental.pallas import tpu as pltpu
from jax.experimental.pallas import tpu_sc as plsc

N = 10000
E = 320000
D = 128

CHUNK = 64
N_CHUNKS = E // CHUNK
NW = 32
CH_PER_W = N_CHUNKS // NW
CH_REM = N_CHUNKS % NW
SLAB_CH = CH_PER_W + (1 if CH_REM else 0)
PIPE = 4
FULL_ITERS = CH_PER_W // PIPE
N_PAD = 10240
ROWS_PER_TILE = N_PAD // 16
DEG_PAD = 10240
DEG_PER_TILE = DEG_PAD // 16


def _make_sc_agg(with_deg):
    mesh = plsc.VectorSubcoreMesh(core_axis_name="c", subcore_axis_name="s")
    out_type = [jax.ShapeDtypeStruct((2, N_PAD, D), jnp.float32)]
    scratch = (
        [pltpu.VMEM((SLAB_CH * CHUNK,), jnp.int32)]
        + [pltpu.VMEM((CHUNK,), jnp.int32) for _ in range(PIPE)]
        + [
            pltpu.VMEM((PIPE, CHUNK, D), jnp.float32),
            pltpu.VMEM_SHARED((N_PAD, D), jnp.float32),
        ]
        + [pltpu.SemaphoreType.DMA for _ in range(4 * PIPE)]
    )
    if with_deg:
        out_type.append(jax.ShapeDtypeStruct((2, DEG_PAD), jnp.float32))
        scratch += [
            pltpu.VMEM((CHUNK,), jnp.float32),
            pltpu.VMEM((DEG_PER_TILE,), jnp.float32),
            pltpu.VMEM_SHARED((DEG_PAD,), jnp.float32),
        ]

    def body(src_hbm, dst_hbm, x_hbm, *refs):
        if with_deg:
            agg_out, deg_out = refs[:2]
            refs = refs[2:]
        else:
            agg_out = refs[0]
            refs = refs[1:]
        src_slab = refs[0]
        dsts = refs[1:1 + PIPE]
        rows_v = refs[1 + PIPE]
        agg_sh = refs[2 + PIPE]
        sems = refs[3 + PIPE:3 + PIPE + 4 * PIPE]
        isem_d = sems[:PIPE]
        gsem = sems[PIPE:2 * PIPE]
        ssem = sems[2 * PIPE:3 * PIPE]
        dsem = sems[3 * PIPE:]
        if with_deg:
            ones_v, zdeg_v, deg_sh = refs[3 + PIPE + 4 * PIPE:]
        cid = lax.axis_index("c")
        sid = lax.axis_index("s")
        wid = sid * 2 + cid
        start_chunk = wid * CH_PER_W + jnp.minimum(wid, CH_REM)
        n_my = CH_PER_W + jnp.where(wid < CH_REM, 1, 0)
        start_load = jnp.minimum(start_chunk, N_CHUNKS - SLAB_CH)
        s_off = start_chunk - start_load
        pltpu.sync_copy(src_hbm.at[pl.ds(start_load * CHUNK, SLAB_CH * CHUNK)],
                        src_slab)

        def zb(i, _):
            rows_v[0, i // 8, pl.ds((i % 8) * 16, 16)] = jnp.zeros((16,), jnp.float32)
            return 0
        lax.fori_loop(0, CHUNK * 8, zb, 0)
        for z in range(ROWS_PER_TILE // CHUNK):
            pltpu.sync_copy(
                rows_v.at[0], agg_sh.at[pl.ds(sid * ROWS_PER_TILE + z * CHUNK, CHUNK)])
        if with_deg:
            def zd(i, _):
                zdeg_v[pl.ds(i * 16, 16)] = jnp.zeros((16,), jnp.float32)
                ones_v[pl.ds((i % (CHUNK // 16)) * 16, 16)] = jnp.ones((16,), jnp.float32)
                return 0
            lax.fori_loop(0, DEG_PER_TILE // 16, zd, 0)
            pltpu.sync_copy(zdeg_v, deg_sh.at[pl.ds(sid * DEG_PER_TILE, DEG_PER_TILE)])
        plsc.subcore_barrier()

        def drain_scatter(k):
            pltpu.make_async_copy(rows_v.at[k], agg_sh.at[dsts[k]], ssem[k]).wait()
            if with_deg:
                pltpu.make_async_copy(ones_v, deg_sh.at[dsts[k]], dsem[k]).wait()

        def src_idx(j):
            return src_slab.at[pl.ds((j + s_off) * CHUNK, CHUNK)]

        def pipe_body(t, _):
            iids = []
            for k in range(PIPE):
                @pl.when(t > 0)
                def _(k=k):
                    drain_scatter(k)
                base = (start_chunk + t * PIPE + k) * CHUNK
                iids.append(
                    pltpu.async_copy(dst_hbm.at[pl.ds(base, CHUNK)], dsts[k], isem_d[k]))
            gds = []
            for k in range(PIPE):
                gds.append(pltpu.async_copy(x_hbm.at[src_idx(t * PIPE + k)],
                                            rows_v.at[k], gsem[k]))
            for k in range(PIPE):
                gds[k].wait()
                iids[k].wait()
                pltpu.async_copy(rows_v.at[k], agg_sh.at[dsts[k]], ssem[k], add=True)
                if with_deg:
                    pltpu.async_copy(ones_v, deg_sh.at[dsts[k]], dsem[k], add=True)
            return 0
        lax.fori_loop(0, FULL_ITERS, pipe_body, 0)
        for k in range(PIPE):
            drain_scatter(k)

        def tail_body(j, _):
            base = (start_chunk + j) * CHUNK
            pltpu.async_copy(dst_hbm.at[pl.ds(base, CHUNK)], dsts[0], isem_d[0]).wait()
            pltpu.async_copy(x_hbm.at[src_idx(j)], rows_v.at[0], gsem[0]).wait()
            pltpu.sync_copy(rows_v.at[0], agg_sh.at[dsts[0]], add=True)
            if with_deg:
                pltpu.sync_copy(ones_v, deg_sh.at[dsts[0]], add=True)
            return 0
        lax.fori_loop(FULL_ITERS * PIPE, n_my, tail_body, 0)
        plsc.subcore_barrier()

        rbase = sid * ROWS_PER_TILE
        pltpu.sync_copy(agg_sh.at[pl.ds(rbase, ROWS_PER_TILE)],
                        agg_out.at[cid, pl.ds(rbase, ROWS_PER_TILE)])
        if with_deg:
            dbase = sid * DEG_PER_TILE
            pltpu.sync_copy(deg_sh.at[pl.ds(dbase, DEG_PER_TILE)],
                            deg_out.at[cid, pl.ds(dbase, DEG_PER_TILE)])

    return pl.kernel(body, out_type=out_type, mesh=mesh, scratch_types=scratch)


_sc_agg_deg = _make_sc_agg(True)
_sc_agg = _make_sc_agg(False)

BR = 1024

_DN = (((1,), (1,)), ((), ()))


def _make_dense_r():
    def body(x_ref, wr_ref, o_ref):
        o_ref[...] = lax.dot_general(x_ref[...], wr_ref[...], _DN,
                                     preferred_element_type=jnp.float32)

    return pl.pallas_call(
        body,
        grid=(N_PAD // BR,),
        in_specs=[
            pl.BlockSpec((BR, D), lambda i: (i, 0)),
            pl.BlockSpec((D, D), lambda i: (0, 0)),
        ],
        out_specs=pl.BlockSpec((BR, D), lambda i: (i, 0)),
        out_shape=jax.ShapeDtypeStruct((N_PAD, D), jnp.float32),
    )


def _make_combine(final_layer):
    def body(p0_ref, p1_ref, d0_ref, d1_ref, xr_ref, wl_ref, b_ref, o_ref):
        deg = jnp.maximum(d0_ref[0, 0] + d1_ref[0, 0], 1.0)
        agg = (p0_ref[0] + p1_ref[0]) / deg
        hl = lax.dot_general(agg, wl_ref[...], _DN, preferred_element_type=jnp.float32)
        h = hl + xr_ref[...] + b_ref[...]
        if final_layer:
            nrm = jnp.sqrt(jnp.sum(h * h, axis=1, keepdims=True))
            o_ref[...] = h / jnp.maximum(nrm, 1e-12)
        else:
            o_ref[...] = jnp.maximum(h, 0.0)

    return pl.pallas_call(
        body,
        grid=(N_PAD // BR,),
        in_specs=[
            pl.BlockSpec((1, BR, D), lambda i: (0, i, 0)),
            pl.BlockSpec((1, BR, D), lambda i: (1, i, 0)),
            pl.BlockSpec((1, 1, BR, 1), lambda i: (0, i, 0, 0)),
            pl.BlockSpec((1, 1, BR, 1), lambda i: (1, i, 0, 0)),
            pl.BlockSpec((BR, D), lambda i: (i, 0)),
            pl.BlockSpec((D, D), lambda i: (0, 0)),
            pl.BlockSpec((1, D), lambda i: (0, 0)),
        ],
        out_specs=pl.BlockSpec((BR, D), lambda i: (i, 0)),
        out_shape=jax.ShapeDtypeStruct((N_PAD, D), jnp.float32),
    )


_dense_r = _make_dense_r()
_combine1 = _make_combine(False)
_combine2 = _make_combine(True)


def kernel(x, edge_index, W1l, b1l, W1r, W2l, b2l, W2r):
    src = edge_index[0]
    dst = edge_index[1]
    xp = jnp.pad(x, ((0, N_PAD - N), (0, 0)))
    part1, degp = _sc_agg_deg(src, dst, xp)
    xr1 = _dense_r(xp, W1r)
    dd = degp.reshape(2, N_PAD // BR, BR, 1)
    h = _combine1(part1, part1, dd, dd, xr1, W1l, b1l.reshape(1, D))
    (part2,) = _sc_agg(src, dst, h)
    xr2 = _dense_r(h, W2r)
    out = _combine2(part2, part2, dd, dd, xr2, W2l, b2l.reshape(1, D))
    return out[:N]


# speedup vs baseline: 1.1030x; 1.0490x over previous
import jax
import jax.numpy as jnp
from jax import lax
from jax.experimental import pallas as pl
from jax.experimental.pallas import tpu as pltpu
from jax.experimental.pallas import tpu_sc as plsc

N = 10000
E = 320000
D = 128

CHUNK = 64
N_CHUNKS = E // CHUNK
NW = 32
CH_PER_W = N_CHUNKS // NW
CH_REM = N_CHUNKS % NW
PIPE = 4
FULL_ITERS = CH_PER_W // PIPE
N_PAD = 10240
ROWS_PER_TILE = N_PAD // 16
DEG_PAD = 10240
DEG_PER_TILE = DEG_PAD // 16


def _make_sc_agg(with_deg):
    mesh = plsc.VectorSubcoreMesh(core_axis_name="c", subcore_axis_name="s")
    out_type = [jax.ShapeDtypeStruct((2, N_PAD, D), jnp.float32)]
    scratch = (
        [pltpu.VMEM((CHUNK,), jnp.int32) for _ in range(PIPE)]
        + [pltpu.VMEM((CHUNK,), jnp.int32) for _ in range(PIPE)]
        + [
            pltpu.VMEM((PIPE, CHUNK, D), jnp.float32),
            pltpu.VMEM_SHARED((N_PAD, D), jnp.float32),
        ]
        + [pltpu.SemaphoreType.DMA for _ in range(5 * PIPE)]
    )
    if with_deg:
        out_type.append(jax.ShapeDtypeStruct((2, DEG_PAD), jnp.float32))
        scratch += [
            pltpu.VMEM((CHUNK,), jnp.float32),
            pltpu.VMEM((DEG_PER_TILE,), jnp.float32),
            pltpu.VMEM_SHARED((DEG_PAD,), jnp.float32),
        ]

    def body(src_hbm, dst_hbm, x_hbm, *refs):
        if with_deg:
            agg_out, deg_out = refs[:2]
            refs = refs[2:]
        else:
            agg_out = refs[0]
            refs = refs[1:]
        srcs = refs[:PIPE]
        dsts = refs[PIPE:2 * PIPE]
        rows_v = refs[2 * PIPE]
        agg_sh = refs[2 * PIPE + 1]
        sems = refs[2 * PIPE + 2:2 * PIPE + 2 + 5 * PIPE]
        isem_s = sems[:PIPE]
        isem_d = sems[PIPE:2 * PIPE]
        gsem = sems[2 * PIPE:3 * PIPE]
        ssem = sems[3 * PIPE:4 * PIPE]
        dsem = sems[4 * PIPE:]
        if with_deg:
            ones_v, zdeg_v, deg_sh = refs[2 * PIPE + 2 + 5 * PIPE:]
        cid = lax.axis_index("c")
        sid = lax.axis_index("s")
        wid = sid * 2 + cid
        start_chunk = wid * CH_PER_W + jnp.minimum(wid, CH_REM)
        n_my = CH_PER_W + jnp.where(wid < CH_REM, 1, 0)

        def zb(i, _):
            rows_v[0, i // 8, pl.ds((i % 8) * 16, 16)] = jnp.zeros((16,), jnp.float32)
            return 0
        lax.fori_loop(0, CHUNK * 8, zb, 0)
        for z in range(ROWS_PER_TILE // CHUNK):
            pltpu.sync_copy(
                rows_v.at[0], agg_sh.at[pl.ds(sid * ROWS_PER_TILE + z * CHUNK, CHUNK)])
        if with_deg:
            def zd(i, _):
                zdeg_v[pl.ds(i * 16, 16)] = jnp.zeros((16,), jnp.float32)
                ones_v[pl.ds((i % (CHUNK // 16)) * 16, 16)] = jnp.ones((16,), jnp.float32)
                return 0
            lax.fori_loop(0, DEG_PER_TILE // 16, zd, 0)
            pltpu.sync_copy(zdeg_v, deg_sh.at[pl.ds(sid * DEG_PER_TILE, DEG_PER_TILE)])
        plsc.subcore_barrier()

        def drain_scatter(k):
            pltpu.make_async_copy(rows_v.at[k], agg_sh.at[dsts[k]], ssem[k]).wait()
            if with_deg:
                pltpu.make_async_copy(ones_v, deg_sh.at[dsts[k]], dsem[k]).wait()

        def pipe_body(t, _):
            iids = []
            for k in range(PIPE):
                @pl.when(t > 0)
                def _(k=k):
                    drain_scatter(k)
                base = (start_chunk + t * PIPE + k) * CHUNK
                iids.append((
                    pltpu.async_copy(src_hbm.at[pl.ds(base, CHUNK)], srcs[k], isem_s[k]),
                    pltpu.async_copy(dst_hbm.at[pl.ds(base, CHUNK)], dsts[k], isem_d[k]),
                ))
            gds = []
            for k in range(PIPE):
                iids[k][0].wait()
                gds.append(pltpu.async_copy(x_hbm.at[srcs[k]], rows_v.at[k], gsem[k]))
            for k in range(PIPE):
                gds[k].wait()
                iids[k][1].wait()
                pltpu.async_copy(rows_v.at[k], agg_sh.at[dsts[k]], ssem[k], add=True)
                if with_deg:
                    pltpu.async_copy(ones_v, deg_sh.at[dsts[k]], dsem[k], add=True)
            return 0
        lax.fori_loop(0, FULL_ITERS, pipe_body, 0)
        for k in range(PIPE):
            drain_scatter(k)

        def tail_body(j, _):
            base = (start_chunk + j) * CHUNK
            pltpu.async_copy(src_hbm.at[pl.ds(base, CHUNK)], srcs[0], isem_s[0]).wait()
            pltpu.async_copy(dst_hbm.at[pl.ds(base, CHUNK)], dsts[0], isem_d[0]).wait()
            pltpu.async_copy(x_hbm.at[srcs[0]], rows_v.at[0], gsem[0]).wait()
            pltpu.sync_copy(rows_v.at[0], agg_sh.at[dsts[0]], add=True)
            if with_deg:
                pltpu.sync_copy(ones_v, deg_sh.at[dsts[0]], add=True)
            return 0
        lax.fori_loop(FULL_ITERS * PIPE, n_my, tail_body, 0)
        plsc.subcore_barrier()

        rbase = sid * ROWS_PER_TILE
        pltpu.sync_copy(agg_sh.at[pl.ds(rbase, ROWS_PER_TILE)],
                        agg_out.at[cid, pl.ds(rbase, ROWS_PER_TILE)])
        if with_deg:
            dbase = sid * DEG_PER_TILE
            pltpu.sync_copy(deg_sh.at[pl.ds(dbase, DEG_PER_TILE)],
                            deg_out.at[cid, pl.ds(dbase, DEG_PER_TILE)])

    return pl.kernel(body, out_type=out_type, mesh=mesh, scratch_types=scratch)


_sc_agg_deg = _make_sc_agg(True)
_sc_agg = _make_sc_agg(False)

BR = 1024

_DN = (((1,), (1,)), ((), ()))


def _make_dense_r():
    def body(x_ref, wr_ref, o_ref):
        o_ref[...] = lax.dot_general(x_ref[...], wr_ref[...], _DN,
                                     preferred_element_type=jnp.float32)

    return pl.pallas_call(
        body,
        grid=(N_PAD // BR,),
        in_specs=[
            pl.BlockSpec((BR, D), lambda i: (i, 0)),
            pl.BlockSpec((D, D), lambda i: (0, 0)),
        ],
        out_specs=pl.BlockSpec((BR, D), lambda i: (i, 0)),
        out_shape=jax.ShapeDtypeStruct((N_PAD, D), jnp.float32),
    )


def _make_combine(final_layer):
    def body(p0_ref, p1_ref, d0_ref, d1_ref, xr_ref, wl_ref, b_ref, o_ref):
        deg = jnp.maximum(d0_ref[0, 0] + d1_ref[0, 0], 1.0)
        agg = (p0_ref[0] + p1_ref[0]) / deg
        hl = lax.dot_general(agg, wl_ref[...], _DN, preferred_element_type=jnp.float32)
        h = hl + xr_ref[...] + b_ref[...]
        if final_layer:
            nrm = jnp.sqrt(jnp.sum(h * h, axis=1, keepdims=True))
            o_ref[...] = h / jnp.maximum(nrm, 1e-12)
        else:
            o_ref[...] = jnp.maximum(h, 0.0)

    return pl.pallas_call(
        body,
        grid=(N_PAD // BR,),
        in_specs=[
            pl.BlockSpec((1, BR, D), lambda i: (0, i, 0)),
            pl.BlockSpec((1, BR, D), lambda i: (1, i, 0)),
            pl.BlockSpec((1, 1, BR, 1), lambda i: (0, i, 0, 0)),
            pl.BlockSpec((1, 1, BR, 1), lambda i: (1, i, 0, 0)),
            pl.BlockSpec((BR, D), lambda i: (i, 0)),
            pl.BlockSpec((D, D), lambda i: (0, 0)),
            pl.BlockSpec((1, D), lambda i: (0, 0)),
        ],
        out_specs=pl.BlockSpec((BR, D), lambda i: (i, 0)),
        out_shape=jax.ShapeDtypeStruct((N_PAD, D), jnp.float32),
    )


_dense_r = _make_dense_r()
_combine1 = _make_combine(False)
_combine2 = _make_combine(True)


def kernel(x, edge_index, W1l, b1l, W1r, W2l, b2l, W2r):
    src = edge_index[0]
    dst = edge_index[1]
    xp = jnp.pad(x, ((0, N_PAD - N), (0, 0)))
    part1, degp = _sc_agg_deg(src, dst, xp)
    xr1 = _dense_r(xp, W1r)
    dd = degp.reshape(2, N_PAD // BR, BR, 1)
    h = _combine1(part1, part1, dd, dd, xr1, W1l, b1l.reshape(1, D))
    (part2,) = _sc_agg(src, dst, h)
    xr2 = _dense_r(h, W2r)
    out = _combine2(part2, part2, dd, dd, xr2, W2l, b2l.reshape(1, D))
    return out[:N]
